# baseline (speedup 1.0000x reference)
import functools

import jax
import jax.numpy as jnp
from jax import lax
from jax.experimental import pallas as pl
from jax.experimental.pallas import tpu as pltpu
from jax.experimental.pallas import tpu_sc as plsc

_NUM_CORES = 2
_NUM_SUBCORES = 16
_NUM_WORKERS = _NUM_CORES * _NUM_SUBCORES

_CHUNK = 512
_TASK = 128


@functools.lru_cache(maxsize=None)
def _gather_call(n_fields, batch, dim):
    n_idx = n_fields * batch
    n_per_w = n_idx // _NUM_WORKERS
    n_chunks = n_per_w // _CHUNK
    tasks_per_chunk = _CHUNK // _TASK
    tasks_per_w = n_chunks * tasks_per_chunk
    n_tc = batch // 128
    n_tr = dim // 8

    mesh = plsc.VectorSubcoreMesh(core_axis_name="c", subcore_axis_name="s")

    @functools.partial(
        pl.kernel,
        mesh=mesh,
        compiler_params=pltpu.CompilerParams(use_tc_tiling_on_sc=False,
                                             needs_layout_passes=False),
        out_type=jax.ShapeDtypeStruct((n_fields, n_tr, n_tc, 8, 128),
                                      jnp.float32),
        scratch_types=[
            pltpu.VMEM((n_per_w,), jnp.int32),
            pltpu.VMEM((2, _CHUNK, dim), jnp.float32),
            pltpu.VMEM((2, n_tr, 8, 128), jnp.float32),
            pltpu.SemaphoreType.DMA,
            pltpu.SemaphoreType.DMA,
            pltpu.SemaphoreType.DMA,
            pltpu.SemaphoreType.DMA,
        ],
    )
    def k(idx_hbm, table_hbm, out_hbm, idx_v, rows_v, blk_v,
          sg0, sg1, sb0, sb1):
        sems_g = (sg0, sg1)
        sems_b = (sb0, sb1)
        wid = lax.axis_index("s") * _NUM_CORES + lax.axis_index("c")
        base = wid * n_per_w
        t0 = wid * tasks_per_w

        lane = jax.lax.iota(jnp.int32, 16)
        colvecs = [jnp.full((16,), d, jnp.int32) for d in range(dim)]

        pltpu.sync_copy(idx_hbm.at[pl.ds(base, n_per_w)], idx_v)

        def start_gather(g, b):
            pltpu.async_copy(
                table_hbm.at[idx_v.at[pl.ds(g * _CHUNK, _CHUNK)]],
                rows_v.at[b], sems_g[b])

        def wait_gather(b):
            pltpu.make_async_copy(
                table_hbm.at[idx_v.at[pl.ds(0, _CHUNK)]],
                rows_v.at[b], sems_g[b]).wait()

        def out_slot(t):
            f = t // n_tc
            tc = lax.rem(t, n_tc)
            return out_hbm.at[f, :, tc]

        def start_block(t, bb):
            pltpu.async_copy(blk_v.at[bb], out_slot(t), sems_b[bb])

        def wait_block(t, bb):
            pltpu.make_async_copy(blk_v.at[bb], out_slot(t),
                                  sems_b[bb]).wait()

        start_gather(0, 0)

        def pair_body(p, carry):
            for bg in range(2):
                g = p * 2 + bg
                wait_gather(bg)

                @pl.when(g + 1 < n_chunks)
                def _():
                    start_gather(g + 1, 1 - bg)

                for j in range(tasks_per_chunk):
                    tl = g * tasks_per_chunk + j
                    bb = j % 2

                    @pl.when(tl >= 2)
                    def _():
                        wait_block(t0 + tl - 2, bb)

                    def c16_body(c16, c2):
                        rowvec = jnp.full((16,), j * _TASK + c16 * 16,
                                          jnp.int32) + lane
                        for d in range(dim):
                            vals = plsc.load_gather(
                                rows_v.at[bg], [rowvec, colvecs[d]])
                            blk_v[bb, d // 8, d % 8,
                                  pl.ds(c16 * 16, 16)] = vals
                        return c2

                    lax.fori_loop(0, _TASK // 16, c16_body, 0)
                    start_block(t0 + tl, bb)
            return carry

        lax.fori_loop(0, n_chunks // 2, pair_body, 0)

        wait_block(t0 + tasks_per_w - 2, 0)
        wait_block(t0 + tasks_per_w - 1, 1)

    return k


def kernel(indices, embedding_weight):
    batch, n_fields = indices.shape
    _, dim = embedding_weight.shape
    idx_flat = indices.T.reshape(batch * n_fields).astype(jnp.int32)
    out5 = _gather_call(n_fields, batch, dim)(idx_flat, embedding_weight)
    return out5.transpose(2, 4, 0, 1, 3).reshape(batch, n_fields, dim)


# speedup vs baseline: 2.5572x; 2.5572x over previous
import functools

import jax
import jax.numpy as jnp
from jax import lax
from jax.experimental import pallas as pl
from jax.experimental.pallas import tpu as pltpu
from jax.experimental.pallas import tpu_sc as plsc

_NUM_CORES = 2
_NUM_SUBCORES = 16
_NUM_WORKERS = _NUM_CORES * _NUM_SUBCORES

_CHUNK = 512
_TASK = 128


@functools.lru_cache(maxsize=None)
def _gather_call(n_fields, batch, dim):
    n_idx = n_fields * batch
    n_per_w = n_idx // _NUM_WORKERS
    n_chunks = n_per_w // _CHUNK
    tasks_per_chunk = _CHUNK // _TASK
    tasks_per_w = n_chunks * tasks_per_chunk
    n_tc = batch // 128
    n_tr = dim // 8

    mesh = plsc.VectorSubcoreMesh(core_axis_name="c", subcore_axis_name="s")

    @functools.partial(
        pl.kernel,
        mesh=mesh,
        compiler_params=pltpu.CompilerParams(use_tc_tiling_on_sc=False,
                                             needs_layout_passes=False),
        out_type=jax.ShapeDtypeStruct((n_fields, n_tr, n_tc, 8, 128),
                                      jnp.float32),
        scratch_types=[
            pltpu.VMEM((n_per_w,), jnp.int32),
            pltpu.VMEM((2, _CHUNK, dim), jnp.float32),
            pltpu.VMEM((2, n_tr, 8, 128), jnp.float32),
            pltpu.SemaphoreType.DMA,
            pltpu.SemaphoreType.DMA,
            pltpu.SemaphoreType.DMA,
            pltpu.SemaphoreType.DMA,
        ],
    )
    def k(idx_hbm, table_hbm, out_hbm, idx_v, rows_v, blk_v,
          sg0, sg1, sb0, sb1):
        sems_g = (sg0, sg1)
        sems_b = (sb0, sb1)
        wid = lax.axis_index("s") * _NUM_CORES + lax.axis_index("c")
        base = wid * n_per_w
        t0 = wid * tasks_per_w

        lane = jax.lax.iota(jnp.int32, 16)
        tr_lo = lax.shift_right_logical(lane, 3)
        tr_hi = tr_lo + 2
        r_all = lax.bitwise_and(lane, 7)

        pltpu.sync_copy(idx_hbm.at[pl.ds(base, n_per_w)], idx_v)

        def start_gather(g, b):
            pltpu.async_copy(
                table_hbm.at[idx_v.at[pl.ds(g * _CHUNK, _CHUNK)]],
                rows_v.at[b], sems_g[b])

        def wait_gather(b):
            pltpu.make_async_copy(
                table_hbm.at[idx_v.at[pl.ds(0, _CHUNK)]],
                rows_v.at[b], sems_g[b]).wait()

        def out_slot(t):
            f = t // n_tc
            tc = lax.rem(t, n_tc)
            return out_hbm.at[f, :, tc]

        def start_block(t, bb):
            pltpu.async_copy(blk_v.at[bb], out_slot(t), sems_b[bb])

        def wait_block(t, bb):
            pltpu.make_async_copy(blk_v.at[bb], out_slot(t),
                                  sems_b[bb]).wait()

        start_gather(0, 0)

        def pair_body(p, carry):
            for bg in range(2):
                g = p * 2 + bg
                wait_gather(bg)

                @pl.when(g + 1 < n_chunks)
                def _():
                    start_gather(g + 1, 1 - bg)

                for j in range(tasks_per_chunk):
                    tl = g * tasks_per_chunk + j
                    bb = j % 2

                    @pl.when(tl >= 2)
                    def _():
                        wait_block(t0 + tl - 2, bb)

                    @functools.partial(plsc.parallel_loop, 0, _TASK,
                                       unroll=4)
                    def _(i):
                        cs = jnp.full((16,), i, jnp.int32)
                        for half, trv in ((0, tr_lo), (1, tr_hi)):
                            vals = rows_v[bg, j * _TASK + i,
                                          pl.ds(half * 16, 16)]
                            plsc.store_scatter(blk_v.at[bb],
                                               [trv, r_all, cs], vals)
                    start_block(t0 + tl, bb)
            return carry

        lax.fori_loop(0, n_chunks // 2, pair_body, 0)

        wait_block(t0 + tasks_per_w - 2, 0)
        wait_block(t0 + tasks_per_w - 1, 1)

    return k


def kernel(indices, embedding_weight):
    batch, n_fields = indices.shape
    _, dim = embedding_weight.shape
    idx_flat = indices.T.reshape(batch * n_fields).astype(jnp.int32)
    out5 = _gather_call(n_fields, batch, dim)(idx_flat, embedding_weight)
    return out5.transpose(2, 4, 0, 1, 3).reshape(batch, n_fields, dim)
